# Initial kernel scaffold; baseline (speedup 1.0000x reference)
import jax
import jax.numpy as jnp
from jax.experimental import pallas as pl


def kernel(input_ids, token_type_ids, position_ids, features, pos_table, type_table, ln_w, ln_b):
    raise NotImplementedError("write your pallas kernel here")


# speedup vs baseline: 1.8760x; 1.8760x over previous
import functools

import jax
import jax.numpy as jnp
from jax import lax
from jax.experimental import pallas as pl
from jax.experimental.pallas import tpu as pltpu
from jax.experimental.pallas import tpu_sc as plsc

B, S, H = 4, 4096, 768
N = B * S
NC, NS = 2, 16
NW = NC * NS
B_PER_W = N // NW
CH = 64
N_CH = B_PER_W // CH
EPS = 1e-12
R = 256


def _sc_gather(table, idx):
    mesh = plsc.VectorSubcoreMesh(
        core_axis_name="c", subcore_axis_name="s",
        num_cores=NC, num_subcores=NS)

    @functools.partial(
        pl.kernel,
        out_type=jax.ShapeDtypeStruct((N, H), jnp.float32),
        mesh=mesh,
        scratch_types=[
            pltpu.VMEM((CH,), jnp.int32),
            pltpu.VMEM((CH, H), jnp.float32),
            pltpu.SemaphoreType.DMA,
        ],
    )
    def gather_kernel(table_hbm, idx_hbm, out_hbm, idx_v, rows_v, sem):
        wid = lax.axis_index("s") * NC + lax.axis_index("c")
        base = wid * B_PER_W

        def body(i, carry):
            off = base + i * CH
            pltpu.sync_copy(idx_hbm.at[pl.ds(off, CH)], idx_v)
            pltpu.async_copy(table_hbm.at[idx_v], rows_v, sem).wait()
            pltpu.sync_copy(rows_v, out_hbm.at[pl.ds(off, CH)])
            return carry

        lax.fori_loop(0, N_CH, body, 0)

    return gather_kernel(table, idx)


def _ln_body(f_ref, g_ref, ttf_ref, tt_ref, w_ref, b_ref, o_ref):
    t0 = tt_ref[0:1, :]
    t1 = tt_ref[1:2, :]
    x = f_ref[...] + g_ref[...] + t0 + ttf_ref[...] * (t1 - t0)
    mean = jnp.mean(x, axis=1, keepdims=True)
    xc = x - mean
    var = jnp.mean(xc * xc, axis=1, keepdims=True)
    o_ref[...] = xc * lax.rsqrt(var + EPS) * w_ref[...] + b_ref[...]


def _tc_ln(feat2d, gathered, ttf, type_table, ln_w, ln_b):
    grid = (N // R,)
    return pl.pallas_call(
        _ln_body,
        grid=grid,
        in_specs=[
            pl.BlockSpec((R, H), lambda i: (i, 0)),
            pl.BlockSpec((R, H), lambda i: (i, 0)),
            pl.BlockSpec((R, 1), lambda i: (i, 0)),
            pl.BlockSpec((2, H), lambda i: (0, 0)),
            pl.BlockSpec((1, H), lambda i: (0, 0)),
            pl.BlockSpec((1, H), lambda i: (0, 0)),
        ],
        out_specs=pl.BlockSpec((R, H), lambda i: (i, 0)),
        out_shape=jax.ShapeDtypeStruct((N, H), jnp.float32),
    )(feat2d, gathered, ttf, type_table, ln_w, ln_b)


def kernel(input_ids, token_type_ids, position_ids, features, pos_table,
           type_table, ln_w, ln_b):
    del input_ids
    pos_flat = position_ids.reshape(N).astype(jnp.int32)
    gathered = _sc_gather(pos_table, pos_flat)
    ttf = token_type_ids.reshape(N, 1).astype(jnp.float32)
    out = _tc_ln(features.reshape(N, H), gathered, ttf, type_table,
                 ln_w.reshape(1, H), ln_b.reshape(1, H))
    return out.reshape(B, S, H)
